# baseline (speedup 1.0000x reference)
import functools

import jax
import jax.numpy as jnp
from jax import lax
from jax.experimental import pallas as pl
from jax.experimental.pallas import tpu as pltpu
from jax.experimental.pallas import tpu_sc as plsc

B_TOTAL = 4 * 256 * 256
D = 512
V = 64
NC = 2
NS = 16
NW = NC * NS
BPW = B_TOTAL // NW
K = 64
NCHUNK = BPW // K
NPAIR = NCHUNK // 2


def _sc_lookup(g_flat, gt_flat, table):
    mesh = plsc.VectorSubcoreMesh(core_axis_name="c", subcore_axis_name="s")

    @functools.partial(
        pl.kernel,
        mesh=mesh,
        out_type=jax.ShapeDtypeStruct((B_TOTAL, D), jnp.float32),
        scratch_types=[
            pltpu.VMEM((BPW + 16,), jnp.int32),
            pltpu.VMEM((V, D), jnp.float32),
            pltpu.VMEM((2, K, D), jnp.float32),
            pltpu.SemaphoreType.DMA,
            pltpu.SemaphoreType.DMA,
        ],
    )
    def body(g_hbm, gt_hbm, table_hbm, out_hbm, idx_v, table_v, rows_v,
             osem0, osem1):
        wid = lax.axis_index("s") * NC + lax.axis_index("c")
        base = wid * BPW
        pltpu.sync_copy(g_hbm.at[pl.ds(base, BPW)], idx_v.at[pl.ds(0, BPW)])
        pltpu.sync_copy(table_hbm, table_v)

        def start_out(c, slot, sem):
            pltpu.async_copy(
                rows_v.at[slot], out_hbm.at[pl.ds(base + c * K, K)], sem)

        def wait_out(slot, sem):
            pltpu.make_async_copy(
                rows_v.at[slot], out_hbm.at[pl.ds(base, K)], sem).wait()

        def fill(c, slot):
            def row(r, carry):
                idx = idx_v[pl.ds(c * K + r, 16)][0]
                for d in range(D // 16):
                    sl = pl.ds(d * 16, 16)
                    rows_v[slot, r, sl] = table_v[idx, sl]
                return carry

            lax.fori_loop(0, K, row, 0)

        def pair(p, carry):
            a = 2 * p

            @pl.when(p > 0)
            def _():
                wait_out(0, osem0)
                wait_out(1, osem1)

            fill(a, 0)
            start_out(a, 0, osem0)
            fill(a + 1, 1)
            start_out(a + 1, 1, osem1)
            return carry

        lax.fori_loop(0, NPAIR, pair, 0)
        wait_out(0, osem0)
        wait_out(1, osem1)

    return body(g_flat, gt_flat, table)


def kernel(graphs, spec_type, normal_type):
    table = jnp.concatenate((spec_type, normal_type), axis=0)
    g_flat = graphs.reshape(B_TOTAL)
    gt_flat = jnp.transpose(graphs, (0, 2, 1)).reshape(B_TOTAL)
    out = _sc_lookup(g_flat, gt_flat, table)
    return out.reshape(4, 256, 256, D)


# speedup vs baseline: 1.5349x; 1.5349x over previous
import functools

import jax
import jax.numpy as jnp
from jax import lax
from jax.experimental import pallas as pl
from jax.experimental.pallas import tpu as pltpu
from jax.experimental.pallas import tpu_sc as plsc

B_TOTAL = 4 * 256 * 256
D = 512
V = 64
NC = 2
NS = 16
NW = NC * NS
BPW = B_TOTAL // NW

KG = 32
NROUND = 32
KF = 32
FPR = 6
G_ROWS = 2 * KG * NROUND
F_BASE = G_ROWS
NFILL = NROUND * FPR


def _sc_lookup(g_flat, gt_flat, table_rep):
    mesh = plsc.VectorSubcoreMesh(core_axis_name="c", subcore_axis_name="s")

    @functools.partial(
        pl.kernel,
        mesh=mesh,
        out_type=jax.ShapeDtypeStruct((B_TOTAL, D), jnp.float32),
        scratch_types=[
            pltpu.VMEM((BPW + 16,), jnp.int32),
            pltpu.VMEM((BPW,), jnp.int32),
            pltpu.VMEM((V, D), jnp.float32),
            pltpu.VMEM((2, KG, D), jnp.float32),
            pltpu.VMEM((2 * KF, D), jnp.float32),
            pltpu.SemaphoreType.DMA,
            pltpu.SemaphoreType.DMA,
            pltpu.SemaphoreType.DMA,
            pltpu.SemaphoreType.DMA,
            pltpu.SemaphoreType.DMA,
        ],
    )
    def body(g_hbm, gt_hbm, table_hbm, out_hbm, idx_v, add_v, table_v, rows_v,
             fb_v, gsem0, gsem1, osem0, osem1, fsem):
        wid = lax.axis_index("s") * NC + lax.axis_index("c")
        base = wid * BPW
        toff = wid * V

        pltpu.sync_copy(g_hbm.at[pl.ds(base, BPW)], idx_v.at[pl.ds(0, BPW)])
        pltpu.sync_copy(gt_hbm.at[pl.ds(base, BPW)], add_v)
        pltpu.sync_copy(table_hbm.at[pl.ds(toff, V)], table_v)

        def add_g(i, carry):
            sl = pl.ds(i * 16, 16)
            idx_v[sl] = idx_v[sl] + add_v[sl] + toff
            return carry

        def add_f(i, carry):
            sl = pl.ds(i * 16, 16)
            idx_v[sl] = idx_v[sl] + add_v[sl]
            return carry

        lax.fori_loop(0, G_ROWS // 16, add_g, 0)
        lax.fori_loop(G_ROWS // 16, BPW // 16, add_f, 0)

        def start_gather(c, slot, sem):
            pltpu.async_copy(
                table_hbm.at[idx_v.at[pl.ds(c * KG, KG)]], rows_v.at[slot], sem)

        def wait_gather(slot, sem):
            pltpu.make_async_copy(
                table_hbm.at[pl.ds(0, KG)], rows_v.at[slot], sem).wait()

        def start_out(c, slot, sem):
            pltpu.async_copy(
                rows_v.at[slot], out_hbm.at[pl.ds(base + c * KG, KG)], sem)

        def wait_out(slot, sem):
            pltpu.make_async_copy(
                rows_v.at[slot], out_hbm.at[pl.ds(base, KG)], sem).wait()

        def fill_chunk(fq, carry):
            @pl.when(fq >= 2)
            def _():
                pltpu.make_async_copy(
                    fb_v.at[pl.ds(0, KF)], out_hbm.at[pl.ds(base, KF)],
                    fsem).wait()

            half = lax.rem(fq, 2) * KF

            def group(g16, carry2):
                row0 = F_BASE + fq * KF + g16 * 16
                iv = idx_v[pl.ds(row0, 16)]
                for l in range(16):
                    idx = iv[l]
                    dst = half + g16 * 16 + l
                    for d in range(D // 16):
                        sl = pl.ds(d * 16, 16)
                        fb_v[dst, sl] = table_v[idx, sl]
                return carry2

            lax.fori_loop(0, KF // 16, group, 0)
            pltpu.async_copy(
                fb_v.at[pl.ds(half, KF)],
                out_hbm.at[pl.ds(base + F_BASE + fq * KF, KF)], fsem)
            return carry

        start_gather(0, 0, gsem0)

        def round_(p, carry):
            a = 2 * p
            b = a + 1
            wait_gather(0, gsem0)

            @pl.when(p > 0)
            def _():
                wait_out(1, osem1)

            start_gather(b, 1, gsem1)
            start_out(a, 0, osem0)
            lax.fori_loop(p * FPR, p * FPR + FPR // 2, fill_chunk, 0)
            wait_gather(1, gsem1)
            wait_out(0, osem0)

            @pl.when(p < NROUND - 1)
            def _():
                start_gather(a + 2, 0, gsem0)

            start_out(b, 1, osem1)
            lax.fori_loop(p * FPR + FPR // 2, (p + 1) * FPR, fill_chunk, 0)
            return carry

        lax.fori_loop(0, NROUND, round_, 0)
        wait_out(1, osem1)
        pltpu.make_async_copy(
            fb_v.at[pl.ds(0, KF)], out_hbm.at[pl.ds(base, KF)], fsem).wait()
        pltpu.make_async_copy(
            fb_v.at[pl.ds(0, KF)], out_hbm.at[pl.ds(base, KF)], fsem).wait()

    return body(g_flat, gt_flat, table_rep)


def kernel(graphs, spec_type, normal_type):
    table = jnp.concatenate((spec_type, normal_type), axis=0)
    table_rep = jnp.tile(table, (NW, 1))
    g_flat = graphs.reshape(B_TOTAL)
    gt_flat = jnp.transpose(graphs, (0, 2, 1)).reshape(B_TOTAL)
    out = _sc_lookup(g_flat, gt_flat, table_rep)
    return out.reshape(4, 256, 256, D)


# speedup vs baseline: 2.5083x; 1.6342x over previous
import functools

import jax
import jax.numpy as jnp
from jax import lax
from jax.experimental import pallas as pl
from jax.experimental.pallas import tpu as pltpu
from jax.experimental.pallas import tpu_sc as plsc

B_TOTAL = 4 * 256 * 256
D = 512
V = 64
NC = 2
NS = 16
NW = NC * NS
BPW = B_TOTAL // NW

KG = 32
NPAIR = 86
G_ROWS = 2 * KG * NPAIR
KF = 16
NFG = (BPW - G_ROWS) // KF
F_BASE = G_ROWS


def _sc_lookup(g_flat, gt_flat, table_rep):
    mesh = plsc.VectorSubcoreMesh(core_axis_name="c", subcore_axis_name="s")

    @functools.partial(
        pl.kernel,
        mesh=mesh,
        out_type=jax.ShapeDtypeStruct((B_TOTAL, D), jnp.float32),
        scratch_types=[
            pltpu.VMEM((BPW + 16,), jnp.int32),
            pltpu.VMEM((BPW,), jnp.int32),
            pltpu.VMEM((V, D), jnp.float32),
            pltpu.VMEM((2, KG, D), jnp.float32),
            pltpu.VMEM((2 * KF, D), jnp.float32),
            pltpu.SemaphoreType.DMA,
            pltpu.SemaphoreType.DMA,
            pltpu.SemaphoreType.DMA,
            pltpu.SemaphoreType.DMA,
            pltpu.SemaphoreType.DMA,
        ],
    )
    def body(g_hbm, gt_hbm, table_hbm, out_hbm, idx_v, add_v, table_v, rows_v,
             fb_v, gsem0, gsem1, osem0, osem1, fsem):
        wid = lax.axis_index("s") * NC + lax.axis_index("c")
        base = wid * BPW
        toff = wid * V

        pltpu.sync_copy(g_hbm.at[pl.ds(base, BPW)], idx_v.at[pl.ds(0, BPW)])
        pltpu.sync_copy(gt_hbm.at[pl.ds(base, BPW)], add_v)
        pltpu.sync_copy(table_hbm.at[pl.ds(toff, V)], table_v)

        def add_g(i, carry):
            sl = pl.ds(i * 16, 16)
            idx_v[sl] = idx_v[sl] + add_v[sl] + toff
            return carry

        def add_f(i, carry):
            sl = pl.ds(i * 16, 16)
            idx_v[sl] = idx_v[sl] + add_v[sl]
            return carry

        lax.fori_loop(0, G_ROWS // 16, add_g, 0)
        lax.fori_loop(G_ROWS // 16, BPW // 16, add_f, 0)

        def start_gather(c, slot, sem):
            pltpu.async_copy(
                table_hbm.at[idx_v.at[pl.ds(c * KG, KG)]], rows_v.at[slot], sem)

        def wait_gather(slot, sem):
            pltpu.make_async_copy(
                table_hbm.at[pl.ds(0, KG)], rows_v.at[slot], sem).wait()

        def start_out(c, slot, sem):
            pltpu.async_copy(
                rows_v.at[slot], out_hbm.at[pl.ds(base + c * KG, KG)], sem)

        def wait_out(slot, sem):
            pltpu.make_async_copy(
                rows_v.at[slot], out_hbm.at[pl.ds(base, KG)], sem).wait()

        def fill_group(fg):
            @pl.when(fg >= 2)
            def _():
                pltpu.make_async_copy(
                    fb_v.at[pl.ds(0, KF)], out_hbm.at[pl.ds(base, KF)],
                    fsem).wait()

            half = lax.rem(fg, 2) * KF
            row0 = F_BASE + fg * KF
            iv = idx_v[pl.ds(row0, 16)]
            for l in range(16):
                idx = iv[l]
                for d in range(D // 16):
                    sl = pl.ds(d * 16, 16)
                    fb_v[half + l, sl] = table_v[idx, sl]
            pltpu.async_copy(
                fb_v.at[pl.ds(half, KF)],
                out_hbm.at[pl.ds(base + row0, KF)], fsem)

        start_gather(0, 0, gsem0)

        def pair(p, carry):
            a = 2 * p
            b = a + 1
            wait_gather(0, gsem0)

            @pl.when(p > 0)
            def _():
                wait_out(1, osem1)

            start_gather(b, 1, gsem1)
            start_out(a, 0, osem0)

            @pl.when(p < NFG // 2)
            def _():
                fill_group(2 * p)

            wait_gather(1, gsem1)
            wait_out(0, osem0)

            @pl.when(p < NPAIR - 1)
            def _():
                start_gather(a + 2, 0, gsem0)

            start_out(b, 1, osem1)

            @pl.when(p < NFG // 2)
            def _():
                fill_group(2 * p + 1)
            return carry

        lax.fori_loop(0, NPAIR, pair, 0)
        wait_out(1, osem1)
        pltpu.make_async_copy(
            fb_v.at[pl.ds(0, KF)], out_hbm.at[pl.ds(base, KF)], fsem).wait()
        pltpu.make_async_copy(
            fb_v.at[pl.ds(0, KF)], out_hbm.at[pl.ds(base, KF)], fsem).wait()

    return body(g_flat, gt_flat, table_rep)


def kernel(graphs, spec_type, normal_type):
    table = jnp.concatenate((spec_type, normal_type), axis=0)
    table_rep = jnp.tile(table, (NW, 1))
    g_flat = graphs.reshape(B_TOTAL)
    gt_flat = jnp.transpose(graphs, (0, 2, 1)).reshape(B_TOTAL)
    out = _sc_lookup(g_flat, gt_flat, table_rep)
    return out.reshape(4, 256, 256, D)
